# Initial kernel scaffold; baseline (speedup 1.0000x reference)
import jax
import jax.numpy as jnp
from jax.experimental import pallas as pl


def kernel(user_id, target_item_id, history_item_id, history_len, user_features, item_features, params):
    raise NotImplementedError("write your pallas kernel here")


# speedup vs baseline: 2.8912x; 2.8912x over previous
import functools

import jax
import jax.numpy as jnp
from jax import lax
from jax.experimental import pallas as pl
from jax.experimental.pallas import tpu as pltpu
from jax.experimental.pallas import tpu_sc as plsc

CHUNK = 128


def _sc_gather(item_table, user_table, hist_idx, user_idx, tgt_idx):
    n_hist = hist_idx.shape[0]
    n_b = user_idx.shape[0]
    e = item_table.shape[1]

    info = plsc.get_sparse_core_info()
    nc, ns = info.num_cores, info.num_subcores
    nw = nc * ns

    hpw = n_hist // nw
    n_chunks = hpw // CHUNK
    bpw = n_b // nw

    hist_idx2 = hist_idx.reshape(nw, n_chunks, CHUNK)
    user_idx2 = user_idx.reshape(nw, 1, bpw)
    tgt_idx2 = tgt_idx.reshape(nw, 1, bpw)

    mesh = plsc.VectorSubcoreMesh(core_axis_name="c", subcore_axis_name="s")

    @functools.partial(
        pl.kernel,
        mesh=mesh,
        compiler_params=pltpu.CompilerParams(use_tc_tiling_on_sc=False),
        out_type=[
            jax.ShapeDtypeStruct((n_hist, e), jnp.float32),
            jax.ShapeDtypeStruct((n_b, e), jnp.float32),
            jax.ShapeDtypeStruct((n_b, e), jnp.float32),
        ],
        scratch_types=[
            pltpu.VMEM((n_chunks, CHUNK), jnp.int32),
            pltpu.VMEM((CHUNK, e), jnp.float32),
            pltpu.VMEM((bpw, e), jnp.float32),
            pltpu.VMEM((1, bpw), jnp.int32),
            pltpu.SemaphoreType.DMA,
        ],
    )
    def gather_kernel(item_tab, user_tab, h_idx, u_idx, t_idx,
                      hist_out, user_out, tgt_out,
                      idx_v, rows_a, rows_b, idx_small, sem):
        wid = lax.axis_index("s") * nc + lax.axis_index("c")
        pltpu.sync_copy(h_idx.at[wid], idx_v)
        hbase = wid * hpw

        def body(j, carry):
            pltpu.async_copy(item_tab.at[idx_v.at[j]], rows_a, sem).wait()
            pltpu.sync_copy(rows_a, hist_out.at[pl.ds(hbase + j * CHUNK, CHUNK)])
            return carry

        lax.fori_loop(0, n_chunks, body, 0)

        base = wid * bpw
        pltpu.sync_copy(u_idx.at[wid], idx_small)
        pltpu.async_copy(user_tab.at[idx_small.at[0]], rows_b, sem).wait()
        pltpu.sync_copy(rows_b, user_out.at[pl.ds(base, bpw)])
        pltpu.sync_copy(t_idx.at[wid], idx_small)
        pltpu.async_copy(item_tab.at[idx_small.at[0]], rows_b, sem).wait()
        pltpu.sync_copy(rows_b, tgt_out.at[pl.ds(base, bpw)])

    return gather_kernel(item_table, user_table, hist_idx2, user_idx2, tgt_idx2)


def _tc_body(bt, t, e,
             hist_ref, te_ref, ue_ref, hl_ref, uf_ref, if_ref,
             wuf_ref, buf_ref, wif_ref, bif_ref,
             wa1_ref, ba1_ref, wa2_ref, ba2_ref, wa3_ref, ba3_ref,
             wu1_ref, bu1_ref, wu2_ref, bu2_ref, wu3_ref,
             wi1_ref, bi1_ref, wi2_ref, bi2_ref, wi3_ref,
             out_ref):
    f32 = jnp.float32
    q = te_ref[...]
    hist = hist_ref[...]
    wa1 = wa1_ref[...]
    a_blk = wa1[0:e]
    b_blk = wa1[e:2 * e]
    c_blk = wa1[2 * e:3 * e]
    d_blk = wa1[3 * e:4 * e]

    qpart = jnp.dot(q, a_blk + c_blk, preferred_element_type=f32) + ba1_ref[...]
    hist3 = hist.reshape(bt, t, e)
    prod = (hist3 * q[:, None, :]).reshape(bt * t, e)
    hp = (jnp.dot(hist, b_blk - c_blk, preferred_element_type=f32)
          + jnp.dot(prod, d_blk, preferred_element_type=f32))
    h1 = jax.nn.sigmoid(hp.reshape(bt, t, e) + qpart[:, None, :])
    h2 = jax.nn.sigmoid(
        jnp.dot(h1.reshape(bt * t, e), wa2_ref[...], preferred_element_type=f32)
        + ba2_ref[...])
    wa3 = wa3_ref[...]
    score = jnp.sum(h2.reshape(bt, t, wa3.shape[1]) * wa3[None, :, :], axis=-1)
    score = score + ba3_ref[0, 0]

    hl = hl_ref[...]
    tmask = lax.broadcasted_iota(jnp.int32, (bt, t), 1) < hl
    score = jnp.where(tmask, score, -1e9)
    m = jnp.max(score, axis=1, keepdims=True)
    ex = jnp.exp(score - m)
    attn = ex / jnp.sum(ex, axis=1, keepdims=True)
    history = jnp.sum(attn[:, :, None] * hist3, axis=1)

    user_feat = jax.nn.sigmoid(
        jnp.dot(uf_ref[...], wuf_ref[...], preferred_element_type=f32) + buf_ref[...])
    item_feat = jax.nn.sigmoid(
        jnp.dot(if_ref[...], wif_ref[...], preferred_element_type=f32) + bif_ref[...])

    cu = jnp.concatenate([ue_ref[...], history, user_feat], axis=1)
    u = jax.nn.relu(jnp.dot(cu, wu1_ref[...], preferred_element_type=f32) + bu1_ref[...])
    u = jax.nn.relu(jnp.dot(u, wu2_ref[...], preferred_element_type=f32) + bu2_ref[...])
    u = jax.nn.relu(jnp.dot(u, wu3_ref[...], preferred_element_type=f32))

    ci = jnp.concatenate([q, item_feat], axis=1)
    it = jax.nn.relu(jnp.dot(ci, wi1_ref[...], preferred_element_type=f32) + bi1_ref[...])
    it = jax.nn.relu(jnp.dot(it, wi2_ref[...], preferred_element_type=f32) + bi2_ref[...])
    it = jax.nn.relu(jnp.dot(it, wi3_ref[...], preferred_element_type=f32))

    out_ref[...] = jnp.sum(u * it, axis=1, keepdims=True)


def _tc_fused(hist_emb, tgt_emb, user_emb, history_len,
              user_features, item_features, p, bt):
    b, e = tgt_emb.shape
    t = hist_emb.shape[0] // b
    grid = (b // bt,)

    def full(shape):
        return pl.BlockSpec(shape, lambda i: (0,) * len(shape))

    in_specs = [
        pl.BlockSpec((bt * t, e), lambda i: (i, 0)),
        pl.BlockSpec((bt, e), lambda i: (i, 0)),
        pl.BlockSpec((bt, e), lambda i: (i, 0)),
        pl.BlockSpec((bt, 1), lambda i: (i, 0)),
        pl.BlockSpec((bt, p['W_uf'].shape[0]), lambda i: (i, 0)),
        pl.BlockSpec((bt, p['W_if'].shape[0]), lambda i: (i, 0)),
        full(p['W_uf'].shape), full((1, e)),
        full(p['W_if'].shape), full((1, e)),
        full(p['Wa1'].shape), full((1, 64)),
        full(p['Wa2'].shape), full((1, 16)),
        full((1, 16)), full((1, 1)),
        full(p['Wu1'].shape), full((1, 200)),
        full(p['Wu2'].shape), full((1, 80)),
        full(p['Wu3'].shape),
        full(p['Wi1'].shape), full((1, 200)),
        full(p['Wi2'].shape), full((1, 80)),
        full(p['Wi3'].shape),
    ]
    out_spec = pl.BlockSpec((bt, 1), lambda i: (i, 0))

    body = functools.partial(_tc_body, bt, t, e)
    return pl.pallas_call(
        body,
        grid=grid,
        in_specs=in_specs,
        out_specs=out_spec,
        out_shape=jax.ShapeDtypeStruct((b, 1), jnp.float32),
    )(
        hist_emb, tgt_emb, user_emb, history_len.reshape(b, 1).astype(jnp.int32),
        user_features, item_features,
        p['W_uf'], p['b_uf'].reshape(1, -1),
        p['W_if'], p['b_if'].reshape(1, -1),
        p['Wa1'], p['ba1'].reshape(1, -1),
        p['Wa2'], p['ba2'].reshape(1, -1),
        p['Wa3'].reshape(1, -1), p['ba3'].reshape(1, 1),
        p['Wu1'], p['bu1'].reshape(1, -1),
        p['Wu2'], p['bu2'].reshape(1, -1),
        p['Wu3'],
        p['Wi1'], p['bi1'].reshape(1, -1),
        p['Wi2'], p['bi2'].reshape(1, -1),
        p['Wi3'],
    )


def kernel(user_id, target_item_id, history_item_id, history_len,
           user_features, item_features, params):
    p = params
    b, t = history_item_id.shape
    uid = user_id.reshape(b).astype(jnp.int32)
    tid = target_item_id.reshape(b).astype(jnp.int32)
    hid = history_item_id.reshape(b * t).astype(jnp.int32)

    hist_emb, user_emb, tgt_emb = _sc_gather(
        p['item_table'], p['user_table'], hid, uid, tid)

    return _tc_fused(hist_emb, tgt_emb, user_emb, history_len,
                     user_features, item_features, p, bt=128)
